# Initial kernel scaffold; baseline (speedup 1.0000x reference)
import jax
import jax.numpy as jnp
from jax.experimental import pallas as pl


def kernel(sequence, segment, token_table, segmt_table, pos_table):
    raise NotImplementedError("write your pallas kernel here")


# speedup vs baseline: 6.0793x; 6.0793x over previous
import functools

import jax
import jax.numpy as jnp
from jax import lax
from jax.experimental import pallas as pl
from jax.experimental.pallas import tpu as pltpu
from jax.experimental.pallas import tpu_sc as plsc

VOCAB = 100000
NUM_SEG = 2
MAX_LEN = 200
EMBED = 128
BATCH = 1024

NC, NS = 2, 16
NW = NC * NS
N = BATCH * MAX_LEN
CHUNK = 128
ROWS_PER_W = N // NW
CHUNKS_PER_W = ROWS_PER_W // CHUNK
LANES = 16


def _tc_prep(segment, segmt_table, pos_table):
    def body(seg_ref, st_ref, pt_ref, cidx_ref, comb_ref):
        pos_ids = lax.broadcasted_iota(jnp.int32, (BATCH, MAX_LEN), 1)
        cidx_ref[...] = seg_ref[...] * MAX_LEN + pos_ids
        comb_ref[...] = st_ref[...][:, None, :] + pt_ref[...][None, :, :]

    return pl.pallas_call(
        body,
        out_shape=[
            jax.ShapeDtypeStruct((BATCH, MAX_LEN), jnp.int32),
            jax.ShapeDtypeStruct((NUM_SEG, MAX_LEN, EMBED), jnp.float32),
        ],
    )(segment, segmt_table, pos_table)


def _sc_gather_add(seq2d, cidx2d, token_table, comb):
    mesh = plsc.VectorSubcoreMesh(
        core_axis_name="c", subcore_axis_name="s",
        num_cores=NC, num_subcores=NS,
    )

    @functools.partial(
        pl.kernel,
        out_type=jax.ShapeDtypeStruct((N, EMBED), jnp.float32),
        mesh=mesh,
        scratch_types=[
            pltpu.VMEM((CHUNKS_PER_W, CHUNK), jnp.int32),
            pltpu.VMEM((CHUNKS_PER_W, CHUNK), jnp.int32),
            pltpu.VMEM((CHUNK, EMBED), jnp.float32),
            pltpu.VMEM((CHUNK, EMBED), jnp.float32),
            pltpu.SemaphoreType.DMA,
            pltpu.SemaphoreType.DMA,
        ],
    )
    def k(seq_hbm, cidx_hbm, tok_hbm, comb_hbm, out_hbm,
          idx_tok, idx_cmb, rows_tok, rows_cmb, sem1, sem2):
        wid = lax.axis_index("s") * NC + lax.axis_index("c")
        obase = pl.multiple_of(wid * ROWS_PER_W, CHUNK)

        pltpu.sync_copy(seq_hbm.at[wid], idx_tok)
        pltpu.sync_copy(cidx_hbm.at[wid], idx_cmb)

        def chunk_body(j, carry):
            cp1 = pltpu.async_copy(tok_hbm.at[idx_tok.at[j]], rows_tok, sem1)
            cp2 = pltpu.async_copy(comb_hbm.at[idx_cmb.at[j]], rows_cmb, sem2)
            cp1.wait()
            cp2.wait()

            def add_body(r, c):
                for kk in range(EMBED // LANES):
                    sl = pl.ds(kk * LANES, LANES)
                    rows_tok[r, sl] = rows_tok[r, sl] + rows_cmb[r, sl]
                return c

            lax.fori_loop(0, CHUNK, add_body, 0)
            obase_j = pl.multiple_of(obase + j * CHUNK, CHUNK)
            pltpu.sync_copy(rows_tok, out_hbm.at[pl.ds(obase_j, CHUNK)])
            return carry

        lax.fori_loop(0, CHUNKS_PER_W, chunk_body, 0)

    return k(seq2d, cidx2d, token_table, comb)


def kernel(sequence, segment, token_table, segmt_table, pos_table):
    seq = sequence.astype(jnp.int32)
    seg = segment.astype(jnp.int32)
    cidx, comb = _tc_prep(seg, segmt_table, pos_table)
    out = _sc_gather_add(
        seq.reshape(NW, CHUNKS_PER_W, CHUNK),
        cidx.reshape(NW, CHUNKS_PER_W, CHUNK),
        token_table,
        comb.reshape(NUM_SEG * MAX_LEN, EMBED),
    )
    return out.reshape(BATCH, MAX_LEN, EMBED)


# speedup vs baseline: 6.5011x; 1.0694x over previous
import functools

import jax
import jax.numpy as jnp
from jax import lax
from jax.experimental import pallas as pl
from jax.experimental.pallas import tpu as pltpu
from jax.experimental.pallas import tpu_sc as plsc

VOCAB = 100000
NUM_SEG = 2
MAX_LEN = 200
EMBED = 128
BATCH = 1024

NC, NS = 2, 16
NW = NC * NS
N = BATCH * MAX_LEN
CHUNK = 128
ROWS_PER_W = N // NW
CHUNKS_PER_W = ROWS_PER_W // CHUNK
LANES = 16


def _tc_prep(segment, segmt_table, pos_table):
    def body(seg_ref, st_ref, pt_ref, cidx_ref, comb_ref):
        pos_ids = lax.broadcasted_iota(jnp.int32, (BATCH, MAX_LEN), 1)
        cidx_ref[...] = seg_ref[...] * MAX_LEN + pos_ids
        comb_ref[...] = st_ref[...][:, None, :] + pt_ref[...][None, :, :]

    return pl.pallas_call(
        body,
        out_shape=[
            jax.ShapeDtypeStruct((BATCH, MAX_LEN), jnp.int32),
            jax.ShapeDtypeStruct((NUM_SEG, MAX_LEN, EMBED), jnp.float32),
        ],
    )(segment, segmt_table, pos_table)


def _sc_gather_add(seq2d, cidx2d, token_table, comb):
    mesh = plsc.VectorSubcoreMesh(
        core_axis_name="c", subcore_axis_name="s",
        num_cores=NC, num_subcores=NS,
    )

    @functools.partial(
        pl.kernel,
        out_type=jax.ShapeDtypeStruct((N, EMBED), jnp.float32),
        mesh=mesh,
        scratch_types=[
            pltpu.VMEM((CHUNKS_PER_W, CHUNK), jnp.int32),
            pltpu.VMEM((CHUNKS_PER_W, CHUNK), jnp.int32),
            pltpu.VMEM((CHUNK, EMBED), jnp.float32),
            pltpu.VMEM((CHUNK, EMBED), jnp.float32),
            pltpu.VMEM((CHUNK, EMBED), jnp.float32),
            pltpu.VMEM((CHUNK, EMBED), jnp.float32),
            pltpu.SemaphoreType.DMA,
            pltpu.SemaphoreType.DMA,
            pltpu.SemaphoreType.DMA,
            pltpu.SemaphoreType.DMA,
        ],
    )
    def k(seq_hbm, cidx_hbm, tok_hbm, comb_hbm, out_hbm,
          idx_tok, idx_cmb, tok0, tok1, cmb0, cmb1, st0, st1, sc0, sc1):
        wid = lax.axis_index("s") * NC + lax.axis_index("c")
        obase = pl.multiple_of(wid * ROWS_PER_W, CHUNK)

        pltpu.sync_copy(seq_hbm.at[wid], idx_tok)
        pltpu.sync_copy(cidx_hbm.at[wid], idx_cmb)

        bufs = ((tok0, cmb0, st0, sc0), (tok1, cmb1, st1, sc1))

        def issue(j, b):
            tok_b, cmb_b, st_b, sc_b = bufs[b]
            pltpu.async_copy(tok_hbm.at[idx_tok.at[j]], tok_b, st_b)
            pltpu.async_copy(comb_hbm.at[idx_cmb.at[j]], cmb_b, sc_b)

        def drain_add_store(j, b):
            tok_b, cmb_b, st_b, sc_b = bufs[b]
            pltpu.make_async_copy(tok_hbm.at[idx_tok.at[j]], tok_b, st_b).wait()
            pltpu.make_async_copy(comb_hbm.at[idx_cmb.at[j]], cmb_b, sc_b).wait()

            def add_body(r, c):
                for kk in range(EMBED // LANES):
                    sl = pl.ds(kk * LANES, LANES)
                    tok_b[r, sl] = tok_b[r, sl] + cmb_b[r, sl]
                return c

            lax.fori_loop(0, CHUNK, add_body, 0)
            obase_j = pl.multiple_of(obase + j * CHUNK, CHUNK)
            pltpu.sync_copy(tok_b, out_hbm.at[pl.ds(obase_j, CHUNK)])

        issue(0, 0)
        issue(1, 1)

        def body(i, carry):
            j0 = i * 2
            drain_add_store(j0, 0)
            issue(j0 + 2, 0)
            drain_add_store(j0 + 1, 1)
            issue(j0 + 3, 1)
            return carry

        lax.fori_loop(0, CHUNKS_PER_W // 2 - 1, body, 0)
        drain_add_store(CHUNKS_PER_W - 2, 0)
        drain_add_store(CHUNKS_PER_W - 1, 1)

    return k(seq2d, cidx2d, token_table, comb)


def kernel(sequence, segment, token_table, segmt_table, pos_table):
    seq = sequence.astype(jnp.int32)
    seg = segment.astype(jnp.int32)
    cidx, comb = _tc_prep(seg, segmt_table, pos_table)
    out = _sc_gather_add(
        seq.reshape(NW, CHUNKS_PER_W, CHUNK),
        cidx.reshape(NW, CHUNKS_PER_W, CHUNK),
        token_table,
        comb.reshape(NUM_SEG * MAX_LEN, EMBED),
    )
    return out.reshape(BATCH, MAX_LEN, EMBED)


# speedup vs baseline: 11.6182x; 1.7871x over previous
import functools

import jax
import jax.numpy as jnp
from jax import lax
from jax.experimental import pallas as pl
from jax.experimental.pallas import tpu as pltpu
from jax.experimental.pallas import tpu_sc as plsc

VOCAB = 100000
NUM_SEG = 2
MAX_LEN = 200
EMBED = 128
BATCH = 1024

NC, NS = 2, 16
NW = NC * NS
N = BATCH * MAX_LEN
CHUNK = 128
ROWS_PER_W = N // NW
CHUNKS_PER_W = ROWS_PER_W // CHUNK
LANES = 16


def _tc_prep(segment, segmt_table, pos_table):
    def body(seg_ref, st_ref, pt_ref, cidx_ref, comb_ref):
        pos_ids = lax.broadcasted_iota(jnp.int32, (BATCH, MAX_LEN), 1)
        cidx_ref[...] = seg_ref[...] * MAX_LEN + pos_ids
        comb_ref[...] = st_ref[...][:, None, :] + pt_ref[...][None, :, :]

    return pl.pallas_call(
        body,
        out_shape=[
            jax.ShapeDtypeStruct((BATCH, MAX_LEN), jnp.int32),
            jax.ShapeDtypeStruct((NUM_SEG, MAX_LEN, EMBED), jnp.float32),
        ],
    )(segment, segmt_table, pos_table)


def _sc_gather_add(seq2d, cidx2d, token_table, comb):
    mesh = plsc.VectorSubcoreMesh(
        core_axis_name="c", subcore_axis_name="s",
        num_cores=NC, num_subcores=NS,
    )

    @functools.partial(
        pl.kernel,
        out_type=jax.ShapeDtypeStruct((N, EMBED), jnp.float32),
        mesh=mesh,
        scratch_types=[
            pltpu.VMEM((CHUNKS_PER_W, CHUNK), jnp.int32),
            pltpu.VMEM((CHUNKS_PER_W, CHUNK), jnp.int32),
            pltpu.VMEM((CHUNK, EMBED), jnp.float32),
            pltpu.VMEM((CHUNK, EMBED), jnp.float32),
            pltpu.VMEM((CHUNK, EMBED), jnp.float32),
            pltpu.VMEM((CHUNK, EMBED), jnp.float32),
            pltpu.VMEM_SHARED((NUM_SEG * MAX_LEN, EMBED), jnp.float32),
            pltpu.SemaphoreType.DMA,
            pltpu.SemaphoreType.DMA,
            pltpu.SemaphoreType.DMA,
            pltpu.SemaphoreType.DMA,
        ],
    )
    def k(seq_hbm, cidx_hbm, tok_hbm, comb_hbm, out_hbm,
          idx_tok, idx_cmb, tok0, tok1, cmb0, cmb1, comb_sp,
          st0, st1, sc0, sc1):
        sid = lax.axis_index("s")
        wid = sid * NC + lax.axis_index("c")
        obase = pl.multiple_of(wid * ROWS_PER_W, CHUNK)

        @pl.when(sid == 0)
        def _():
            pltpu.sync_copy(comb_hbm, comb_sp)

        pltpu.sync_copy(seq_hbm.at[wid], idx_tok)
        pltpu.sync_copy(cidx_hbm.at[wid], idx_cmb)
        plsc.subcore_barrier()

        bufs = ((tok0, cmb0, st0, sc0), (tok1, cmb1, st1, sc1))

        def issue(j, b):
            tok_b, cmb_b, st_b, sc_b = bufs[b]
            pltpu.async_copy(tok_hbm.at[idx_tok.at[j]], tok_b, st_b)
            pltpu.async_copy(comb_sp.at[idx_cmb.at[j]], cmb_b, sc_b)

        def drain_add_store(j, b):
            tok_b, cmb_b, st_b, sc_b = bufs[b]
            pltpu.make_async_copy(tok_hbm.at[idx_tok.at[j]], tok_b, st_b).wait()
            pltpu.make_async_copy(comb_sp.at[idx_cmb.at[j]], cmb_b, sc_b).wait()

            def add_body(r, c):
                for kk in range(EMBED // LANES):
                    sl = pl.ds(kk * LANES, LANES)
                    tok_b[r, sl] = tok_b[r, sl] + cmb_b[r, sl]
                return c

            lax.fori_loop(0, CHUNK, add_body, 0)
            obase_j = pl.multiple_of(obase + j * CHUNK, CHUNK)
            pltpu.sync_copy(tok_b, out_hbm.at[pl.ds(obase_j, CHUNK)])

        issue(0, 0)
        issue(1, 1)

        def body(i, carry):
            j0 = i * 2
            drain_add_store(j0, 0)
            issue(j0 + 2, 0)
            drain_add_store(j0 + 1, 1)
            issue(j0 + 3, 1)
            return carry

        lax.fori_loop(0, CHUNKS_PER_W // 2 - 1, body, 0)
        drain_add_store(CHUNKS_PER_W - 2, 0)
        drain_add_store(CHUNKS_PER_W - 1, 1)

    return k(seq2d, cidx2d, token_table, comb)


def kernel(sequence, segment, token_table, segmt_table, pos_table):
    seq = sequence.astype(jnp.int32)
    seg = segment.astype(jnp.int32)
    cidx, comb = _tc_prep(seg, segmt_table, pos_table)
    out = _sc_gather_add(
        seq.reshape(NW, CHUNKS_PER_W, CHUNK),
        cidx.reshape(NW, CHUNKS_PER_W, CHUNK),
        token_table,
        comb.reshape(NUM_SEG * MAX_LEN, EMBED),
    )
    return out.reshape(BATCH, MAX_LEN, EMBED)


# speedup vs baseline: 12.4138x; 1.0685x over previous
import functools

import jax
import jax.numpy as jnp
from jax import lax
from jax.experimental import pallas as pl
from jax.experimental.pallas import tpu as pltpu
from jax.experimental.pallas import tpu_sc as plsc

VOCAB = 100000
NUM_SEG = 2
MAX_LEN = 200
EMBED = 128
BATCH = 1024

NC, NS = 2, 16
NW = NC * NS
N = BATCH * MAX_LEN
CHUNK = 128
ROWS_PER_W = N // NW
CHUNKS_PER_W = ROWS_PER_W // CHUNK
LANES = 16


def _tc_prep(segment, segmt_table, pos_table):
    def body(seg_ref, st_ref, pt_ref, cidx_ref, comb_ref):
        pos_ids = lax.broadcasted_iota(jnp.int32, (BATCH, MAX_LEN), 1)
        cidx_ref[...] = seg_ref[...] * MAX_LEN + pos_ids
        comb_ref[...] = st_ref[...][:, None, :] + pt_ref[...][None, :, :]

    return pl.pallas_call(
        body,
        out_shape=[
            jax.ShapeDtypeStruct((BATCH, MAX_LEN), jnp.int32),
            jax.ShapeDtypeStruct((NUM_SEG, MAX_LEN, EMBED), jnp.float32),
        ],
    )(segment, segmt_table, pos_table)


def _sc_gather_add(seq2d, cidx2d, token_table, comb):
    mesh = plsc.VectorSubcoreMesh(
        core_axis_name="c", subcore_axis_name="s",
        num_cores=NC, num_subcores=NS,
    )

    @functools.partial(
        pl.kernel,
        out_type=jax.ShapeDtypeStruct((N, EMBED), jnp.float32),
        mesh=mesh,
        scratch_types=[
            pltpu.VMEM((CHUNKS_PER_W, CHUNK), jnp.int32),
            pltpu.VMEM((CHUNKS_PER_W, CHUNK), jnp.int32),
            pltpu.VMEM((CHUNK, EMBED), jnp.float32),
            pltpu.VMEM((CHUNK, EMBED), jnp.float32),
            pltpu.VMEM((CHUNK, EMBED), jnp.float32),
            pltpu.VMEM((CHUNK, EMBED), jnp.float32),
            pltpu.VMEM_SHARED((NUM_SEG * MAX_LEN, EMBED), jnp.float32),
            pltpu.SemaphoreType.DMA,
            pltpu.SemaphoreType.DMA,
            pltpu.SemaphoreType.DMA,
            pltpu.SemaphoreType.DMA,
        ],
    )
    def k(seq_hbm, cidx_hbm, tok_hbm, comb_hbm, out_hbm,
          idx_tok, idx_cmb, tok0, tok1, cmb0, cmb1, comb_sp,
          st0, st1, sc0, sc1):
        sid = lax.axis_index("s")
        wid = sid * NC + lax.axis_index("c")
        obase = pl.multiple_of(wid * ROWS_PER_W, CHUNK)

        @pl.when(sid == 0)
        def _():
            pltpu.sync_copy(comb_hbm, comb_sp)

        pltpu.sync_copy(seq_hbm.at[wid], idx_tok)
        pltpu.sync_copy(cidx_hbm.at[wid], idx_cmb)
        plsc.subcore_barrier()

        bufs = ((tok0, cmb0, st0, sc0), (tok1, cmb1, st1, sc1))

        def issue(j, b):
            tok_b, cmb_b, st_b, sc_b = bufs[b]
            pltpu.async_copy(tok_hbm.at[idx_tok.at[j]], tok_b, st_b)

        def drain_add_store(j, b):
            tok_b, cmb_b, st_b, sc_b = bufs[b]
            pltpu.make_async_copy(tok_hbm.at[idx_tok.at[j]], tok_b, st_b).wait()
            pltpu.async_copy(comb_sp.at[idx_cmb.at[j]], tok_b, sc_b,
                             add=True).wait()
            obase_j = pl.multiple_of(obase + j * CHUNK, CHUNK)
            pltpu.sync_copy(tok_b, out_hbm.at[pl.ds(obase_j, CHUNK)])

        issue(0, 0)
        issue(1, 1)

        def body(i, carry):
            j0 = i * 2
            drain_add_store(j0, 0)
            issue(j0 + 2, 0)
            drain_add_store(j0 + 1, 1)
            issue(j0 + 3, 1)
            return carry

        lax.fori_loop(0, CHUNKS_PER_W // 2 - 1, body, 0)
        drain_add_store(CHUNKS_PER_W - 2, 0)
        drain_add_store(CHUNKS_PER_W - 1, 1)

    return k(seq2d, cidx2d, token_table, comb)


def kernel(sequence, segment, token_table, segmt_table, pos_table):
    seq = sequence.astype(jnp.int32)
    seg = segment.astype(jnp.int32)
    cidx, comb = _tc_prep(seg, segmt_table, pos_table)
    out = _sc_gather_add(
        seq.reshape(NW, CHUNKS_PER_W, CHUNK),
        cidx.reshape(NW, CHUNKS_PER_W, CHUNK),
        token_table,
        comb.reshape(NUM_SEG * MAX_LEN, EMBED),
    )
    return out.reshape(BATCH, MAX_LEN, EMBED)


# speedup vs baseline: 13.8216x; 1.1134x over previous
import functools

import jax
import jax.numpy as jnp
from jax import lax
from jax.experimental import pallas as pl
from jax.experimental.pallas import tpu as pltpu
from jax.experimental.pallas import tpu_sc as plsc

VOCAB = 100000
NUM_SEG = 2
MAX_LEN = 200
EMBED = 128
BATCH = 1024

NC, NS = 2, 16
NW = NC * NS
N = BATCH * MAX_LEN
CHUNK = 128
ROWS_PER_W = N // NW
CHUNKS_PER_W = ROWS_PER_W // CHUNK
LANES = 16


def _tc_prep(segment, segmt_table, pos_table):
    def body(seg_ref, st_ref, pt_ref, cidx_ref, comb_ref):
        pos_ids = lax.broadcasted_iota(jnp.int32, (BATCH, MAX_LEN), 1)
        cidx_ref[...] = seg_ref[...] * MAX_LEN + pos_ids
        comb_ref[...] = st_ref[...][:, None, :] + pt_ref[...][None, :, :]

    return pl.pallas_call(
        body,
        out_shape=[
            jax.ShapeDtypeStruct((BATCH, MAX_LEN), jnp.int32),
            jax.ShapeDtypeStruct((NUM_SEG, MAX_LEN, EMBED), jnp.float32),
        ],
    )(segment, segmt_table, pos_table)


def _sc_gather_add(seq2d, cidx2d, token_table, comb):
    mesh = plsc.VectorSubcoreMesh(
        core_axis_name="c", subcore_axis_name="s",
        num_cores=NC, num_subcores=NS,
    )

    NBUF = 4

    @functools.partial(
        pl.kernel,
        out_type=jax.ShapeDtypeStruct((N, EMBED), jnp.float32),
        mesh=mesh,
        scratch_types=(
            [pltpu.VMEM((CHUNKS_PER_W, CHUNK), jnp.int32)] * 2
            + [pltpu.VMEM((CHUNK, EMBED), jnp.float32)] * NBUF
            + [pltpu.VMEM_SHARED((NUM_SEG * MAX_LEN, EMBED), jnp.float32)]
            + [pltpu.SemaphoreType.DMA] * (3 * NBUF)
        ),
    )
    def k(seq_hbm, cidx_hbm, tok_hbm, comb_hbm, out_hbm,
          idx_tok, idx_cmb, *rest):
        rows = rest[:NBUF]
        comb_sp = rest[NBUF]
        sg = rest[NBUF + 1:NBUF + 1 + NBUF]
        sa = rest[NBUF + 1 + NBUF:NBUF + 1 + 2 * NBUF]
        so = rest[NBUF + 1 + 2 * NBUF:]

        sid = lax.axis_index("s")
        wid = sid * NC + lax.axis_index("c")
        obase = pl.multiple_of(wid * ROWS_PER_W, CHUNK)

        @pl.when(sid == 0)
        def _():
            pltpu.sync_copy(comb_hbm, comb_sp)

        pltpu.sync_copy(seq_hbm.at[wid], idx_tok)
        pltpu.sync_copy(cidx_hbm.at[wid], idx_cmb)
        plsc.subcore_barrier()

        def out_at(j):
            return out_hbm.at[pl.ds(pl.multiple_of(obase + j * CHUNK, CHUNK),
                                    CHUNK)]

        def issue_tok(j, b):
            pltpu.async_copy(tok_hbm.at[idx_tok.at[j]], rows[b], sg[b])

        def wait_tok(j, b):
            pltpu.make_async_copy(tok_hbm.at[idx_tok.at[j]], rows[b],
                                  sg[b]).wait()

        def issue_add(j, b):
            pltpu.async_copy(comb_sp.at[idx_cmb.at[j]], rows[b], sa[b],
                             add=True)

        def wait_add(j, b):
            pltpu.make_async_copy(comb_sp.at[idx_cmb.at[j]], rows[b],
                                  sa[b]).wait()

        def issue_store(j, b):
            pltpu.async_copy(rows[b], out_at(j), so[b])

        def wait_store(j, b):
            pltpu.make_async_copy(rows[b], out_at(j), so[b]).wait()

        def phase(j, b, first=False, second=False, issue_next=True):
            wait_tok(j, b)
            issue_add(j, b)
            if not first:
                wait_add(j - 1, (b - 1) % NBUF)
                issue_store(j - 1, (b - 1) % NBUF)
            if not (first or second):
                wait_store(j - 2, (b - 2) % NBUF)
            if issue_next:
                issue_tok(j + 2, (b + 2) % NBUF)

        issue_tok(0, 0)
        issue_tok(1, 1)
        phase(0, 0, first=True)
        phase(1, 1, second=True)

        def body(i, carry):
            j0 = 2 + i * NBUF
            for t in range(NBUF):
                phase(j0 + t, (2 + t) % NBUF)
            return carry

        lax.fori_loop(0, (CHUNKS_PER_W - 2 - NBUF) // NBUF, body, 0)
        for j in range(CHUNKS_PER_W - NBUF, CHUNKS_PER_W):
            phase(j, j % NBUF, issue_next=(j + 2 < CHUNKS_PER_W))
        j_last = CHUNKS_PER_W - 1
        wait_add(j_last, j_last % NBUF)
        issue_store(j_last, j_last % NBUF)
        wait_store(j_last - 1, (j_last - 1) % NBUF)
        wait_store(j_last, j_last % NBUF)

    return k(seq2d, cidx2d, token_table, comb)


def kernel(sequence, segment, token_table, segmt_table, pos_table):
    seq = sequence.astype(jnp.int32)
    seg = segment.astype(jnp.int32)
    cidx, comb = _tc_prep(seg, segmt_table, pos_table)
    out = _sc_gather_add(
        seq.reshape(NW, CHUNKS_PER_W, CHUNK),
        cidx.reshape(NW, CHUNKS_PER_W, CHUNK),
        token_table,
        comb.reshape(NUM_SEG * MAX_LEN, EMBED),
    )
    return out.reshape(BATCH, MAX_LEN, EMBED)


# speedup vs baseline: 14.1974x; 1.0272x over previous
import functools

import jax
import jax.numpy as jnp
from jax import lax
from jax.experimental import pallas as pl
from jax.experimental.pallas import tpu as pltpu
from jax.experimental.pallas import tpu_sc as plsc

VOCAB = 100000
NUM_SEG = 2
MAX_LEN = 200
EMBED = 128
BATCH = 1024

NC, NS = 2, 16
NW = NC * NS
N = BATCH * MAX_LEN
CHUNK = 128
ROWS_PER_W = N // NW
CHUNKS_PER_W = ROWS_PER_W // CHUNK
LANES = 16
NBUF = 4
PIECE = 40


def _sc_embed(seq3d, seg3d, token_table, segmt_table, pos_table):
    mesh = plsc.VectorSubcoreMesh(
        core_axis_name="c", subcore_axis_name="s",
        num_cores=NC, num_subcores=NS,
    )

    @functools.partial(
        pl.kernel,
        out_type=jax.ShapeDtypeStruct((N, EMBED), jnp.float32),
        mesh=mesh,
        scratch_types=(
            [pltpu.VMEM((CHUNKS_PER_W, CHUNK), jnp.int32)] * 2
            + [pltpu.VMEM((CHUNK, EMBED), jnp.float32)] * NBUF
            + [pltpu.VMEM((NUM_SEG, EMBED), jnp.float32)]
            + [pltpu.VMEM_SHARED((NUM_SEG * MAX_LEN, EMBED), jnp.float32)]
            + [pltpu.SemaphoreType.DMA] * (3 * NBUF)
        ),
    )
    def k(seq_hbm, seg_hbm, tok_hbm, st_hbm, pos_hbm, out_hbm,
          idx_tok, idx_cmb, *rest):
        rows = rest[:NBUF]
        segv = rest[NBUF]
        comb_sp = rest[NBUF + 1]
        sg = rest[NBUF + 2:NBUF + 2 + NBUF]
        sa = rest[NBUF + 2 + NBUF:NBUF + 2 + 2 * NBUF]
        so = rest[NBUF + 2 + 2 * NBUF:]

        sid = lax.axis_index("s")
        wid = sid * NC + lax.axis_index("c")
        obase = pl.multiple_of(wid * ROWS_PER_W, CHUNK)

        pltpu.sync_copy(seq_hbm.at[wid], idx_tok)
        pltpu.sync_copy(seg_hbm.at[wid], idx_cmb)

        def out_at(j):
            return out_hbm.at[pl.ds(pl.multiple_of(obase + j * CHUNK, CHUNK),
                                    CHUNK)]

        def issue_tok(j, b):
            pltpu.async_copy(tok_hbm.at[idx_tok.at[j]], rows[b], sg[b])

        def wait_tok(j, b):
            pltpu.make_async_copy(tok_hbm.at[idx_tok.at[j]], rows[b],
                                  sg[b]).wait()

        def issue_add(j, b):
            pltpu.async_copy(comb_sp.at[idx_cmb.at[j]], rows[b], sa[b],
                             add=True)

        def wait_add(j, b):
            pltpu.make_async_copy(comb_sp.at[idx_cmb.at[j]], rows[b],
                                  sa[b]).wait()

        def issue_store(j, b):
            pltpu.async_copy(rows[b], out_at(j), so[b])

        def wait_store(j, b):
            pltpu.make_async_copy(rows[b], out_at(j), so[b]).wait()

        issue_tok(0, 0)
        issue_tok(1, 1)

        @pl.when(sid < MAX_LEN * NUM_SEG // PIECE)
        def _():
            pltpu.sync_copy(st_hbm, segv)
            poff = pl.multiple_of(lax.rem(sid, MAX_LEN // PIECE) * PIECE, 8)
            s2 = sid // (MAX_LEN // PIECE)
            pltpu.sync_copy(pos_hbm.at[pl.ds(poff, PIECE)],
                            rows[3].at[pl.ds(0, PIECE)])

            def add_body(r, c):
                for g in range(EMBED // LANES):
                    sl = pl.ds(g * LANES, LANES)
                    rows[3][r, sl] = rows[3][r, sl] + segv[s2, sl]
                return c

            lax.fori_loop(0, PIECE, add_body, 0)
            coff = pl.multiple_of(sid * PIECE, 8)
            pltpu.sync_copy(rows[3].at[pl.ds(0, PIECE)],
                            comb_sp.at[pl.ds(coff, PIECE)])

        def cidx_body(r, c):
            for g in range(EMBED // LANES):
                sl = pl.ds(g * LANES, LANES)
                flat = r * CHUNK + g * LANES + lax.broadcasted_iota(
                    jnp.int32, (LANES,), 0)
                idx_cmb[r, sl] = (idx_cmb[r, sl] * MAX_LEN
                                  + lax.rem(flat, MAX_LEN))
            return c

        lax.fori_loop(0, CHUNKS_PER_W, cidx_body, 0)
        plsc.subcore_barrier()

        def phase(j, b, first=False, second=False, issue_next=True):
            wait_tok(j, b)
            issue_add(j, b)
            if not first:
                wait_add(j - 1, (b - 1) % NBUF)
                issue_store(j - 1, (b - 1) % NBUF)
            if not (first or second):
                wait_store(j - 2, (b - 2) % NBUF)
            if issue_next:
                issue_tok(j + 2, (b + 2) % NBUF)

        phase(0, 0, first=True)
        phase(1, 1, second=True)

        def body(i, carry):
            j0 = 2 + i * NBUF
            for t in range(NBUF):
                phase(j0 + t, (2 + t) % NBUF)
            return carry

        lax.fori_loop(0, (CHUNKS_PER_W - 2 - NBUF) // NBUF, body, 0)
        for j in range(CHUNKS_PER_W - NBUF, CHUNKS_PER_W):
            phase(j, j % NBUF, issue_next=(j + 2 < CHUNKS_PER_W))
        j_last = CHUNKS_PER_W - 1
        wait_add(j_last, j_last % NBUF)
        issue_store(j_last, j_last % NBUF)
        wait_store(j_last - 1, (j_last - 1) % NBUF)
        wait_store(j_last, j_last % NBUF)

    return k(seq3d, seg3d, token_table, segmt_table, pos_table)


def kernel(sequence, segment, token_table, segmt_table, pos_table):
    seq = sequence.astype(jnp.int32).reshape(NW, CHUNKS_PER_W, CHUNK)
    seg = segment.astype(jnp.int32).reshape(NW, CHUNKS_PER_W, CHUNK)
    out = _sc_embed(seq, seg, token_table, segmt_table, pos_table)
    return out.reshape(BATCH, MAX_LEN, EMBED)
